# baseline (speedup 1.0000x reference)
import functools

import jax
import jax.numpy as jnp
from jax import lax
from jax.experimental import pallas as pl
from jax.experimental.pallas import tpu as pltpu
from jax.experimental.pallas import tpu_sc as plsc

_B = 4096
_K = 512
_D = 64
_RB = 512
_NB = _B // _RB


def _assign_body(x_ref, ct_ref, out_ref):
    x = x_ref[...]
    ct = ct_ref[...]
    cn = jnp.sum(ct * ct, axis=0, keepdims=True)
    xc = lax.dot_general(
        x, ct, (((1,), (0,)), ((), ())),
        preferred_element_type=jnp.float32,
        precision=lax.Precision.HIGHEST,
    )
    scores = cn - 2.0 * xc
    rowmin = jnp.min(scores, axis=1, keepdims=True)
    ids = lax.broadcasted_iota(jnp.int32, scores.shape, 1)
    idx = jnp.min(jnp.where(scores == rowmin, ids, _K), axis=1, keepdims=True)
    out_ref[...] = idx


def _assignments(inputs, clusters_t):
    out = pl.pallas_call(
        _assign_body,
        grid=(_NB,),
        in_specs=[
            pl.BlockSpec((_RB, _D), lambda i: (i, 0)),
            pl.BlockSpec((_D, _K), lambda i: (0, 0)),
        ],
        out_specs=pl.BlockSpec((_RB, 1), lambda i: (i, 0)),
        out_shape=jax.ShapeDtypeStruct((_B, 1), jnp.int32),
    )(inputs, clusters_t)
    return out.reshape(_B)


_NC = 1
_NS = 16
_NW = _NC * _NS
_BPW = _B // _NW


_CH = _BPW // 2


@functools.cache
def _gather_rows():
    @functools.partial(
        pl.kernel,
        mesh=plsc.VectorSubcoreMesh(core_axis_name="c", subcore_axis_name="s",
                                    num_cores=_NC),
        out_type=jax.ShapeDtypeStruct((_B, _D), jnp.float32),
        scratch_types=[
            pltpu.VMEM((_CH,), jnp.int32),
            pltpu.VMEM((_CH,), jnp.int32),
            pltpu.VMEM((_CH, _D), jnp.float32),
            pltpu.VMEM((_CH, _D), jnp.float32),
            pltpu.SemaphoreType.DMA,
            pltpu.SemaphoreType.DMA,
            pltpu.SemaphoreType.DMA,
            pltpu.SemaphoreType.DMA,
        ],
        compiler_params=pltpu.CompilerParams(use_tc_tiling_on_sc=False),
    )
    def gather_k(table_hbm, idx_hbm, out_hbm,
                 idx0_v, idx1_v, rows0_v, rows1_v, g0, g1, s0, s1):
        wid = lax.axis_index("s") * _NC + lax.axis_index("c")
        base = wid * _BPW
        pltpu.sync_copy(idx_hbm.at[pl.ds(base, _CH)], idx0_v)
        gather0 = pltpu.async_copy(table_hbm.at[idx0_v], rows0_v, g0)
        pltpu.sync_copy(idx_hbm.at[pl.ds(base + _CH, _CH)], idx1_v)
        gather1 = pltpu.async_copy(table_hbm.at[idx1_v], rows1_v, g1)
        gather0.wait()
        store0 = pltpu.async_copy(rows0_v, out_hbm.at[pl.ds(base, _CH)], s0)
        gather1.wait()
        store1 = pltpu.async_copy(rows1_v, out_hbm.at[pl.ds(base + _CH, _CH)], s1)
        store0.wait()
        store1.wait()

    return gather_k


def kernel(inputs, clusters):
    assignments = _assignments(inputs, clusters.T)
    return _gather_rows()(clusters, assignments)


# speedup vs baseline: 1.1415x; 1.1415x over previous
import functools

import jax
import jax.numpy as jnp
from jax import lax
from jax.experimental import pallas as pl
from jax.experimental.pallas import tpu as pltpu
from jax.experimental.pallas import tpu_sc as plsc

_B = 4096
_K = 512
_D = 64
_RB = 512
_NB = _B // _RB


def _assign_body(x_ref, ct_ref, out_ref):
    x = x_ref[...]
    ct = ct_ref[...]
    cn = jnp.sum(ct * ct, axis=0, keepdims=True)
    xc = lax.dot_general(
        x, ct, (((1,), (0,)), ((), ())),
        preferred_element_type=jnp.float32,
        precision=lax.Precision.HIGHEST,
    )
    scores = cn - 2.0 * xc
    rowmin = jnp.min(scores, axis=1, keepdims=True)
    ids = lax.broadcasted_iota(jnp.int32, scores.shape, 1)
    idx = jnp.min(jnp.where(scores == rowmin, ids, _K), axis=1, keepdims=True)
    out_ref[...] = idx


def _assignments(inputs, clusters_t):
    out = pl.pallas_call(
        _assign_body,
        grid=(_NB,),
        in_specs=[
            pl.BlockSpec((_RB, _D), lambda i: (i, 0)),
            pl.BlockSpec((_D, _K), lambda i: (0, 0)),
        ],
        out_specs=pl.BlockSpec((_RB, 1), lambda i: (i, 0)),
        out_shape=jax.ShapeDtypeStruct((_B, 1), jnp.int32),
        compiler_params=pltpu.CompilerParams(
            allow_input_fusion=[False, True]),
    )(inputs, clusters_t)
    return out.reshape(_B)


_NC = 1
_NS = 16
_NW = _NC * _NS
_BPW = _B // _NW


_CH = _BPW // 2


@functools.cache
def _gather_rows():
    @functools.partial(
        pl.kernel,
        mesh=plsc.VectorSubcoreMesh(core_axis_name="c", subcore_axis_name="s",
                                    num_cores=_NC),
        out_type=jax.ShapeDtypeStruct((_B, _D), jnp.float32),
        scratch_types=[
            pltpu.VMEM((_CH,), jnp.int32),
            pltpu.VMEM((_CH,), jnp.int32),
            pltpu.VMEM((_CH, _D), jnp.float32),
            pltpu.VMEM((_CH, _D), jnp.float32),
            pltpu.SemaphoreType.DMA,
            pltpu.SemaphoreType.DMA,
            pltpu.SemaphoreType.DMA,
            pltpu.SemaphoreType.DMA,
        ],
        compiler_params=pltpu.CompilerParams(use_tc_tiling_on_sc=False),
    )
    def gather_k(table_hbm, idx_hbm, out_hbm,
                 idx0_v, idx1_v, rows0_v, rows1_v, g0, g1, s0, s1):
        wid = lax.axis_index("s") * _NC + lax.axis_index("c")
        base = wid * _BPW
        pltpu.sync_copy(idx_hbm.at[pl.ds(base, _CH)], idx0_v)
        gather0 = pltpu.async_copy(table_hbm.at[idx0_v], rows0_v, g0)
        pltpu.sync_copy(idx_hbm.at[pl.ds(base + _CH, _CH)], idx1_v)
        gather1 = pltpu.async_copy(table_hbm.at[idx1_v], rows1_v, g1)
        gather0.wait()
        store0 = pltpu.async_copy(rows0_v, out_hbm.at[pl.ds(base, _CH)], s0)
        gather1.wait()
        store1 = pltpu.async_copy(rows1_v, out_hbm.at[pl.ds(base + _CH, _CH)], s1)
        store0.wait()
        store1.wait()

    return gather_k


def kernel(inputs, clusters):
    assignments = _assignments(inputs, clusters.T)
    return _gather_rows()(clusters, assignments)
